# baseline (speedup 1.0000x reference)
import functools

import jax
import jax.numpy as jnp
from jax import lax
from jax.experimental import pallas as pl
from jax.experimental.pallas import tpu as pltpu
from jax.experimental.pallas import tpu_sc as plsc

B, N, M = 8, 8192, 1024
C1, C2 = 64, 128
C_IN, C_OUT = 192, 128

TN = 4096
NB = N // TN


def _knn_body(unknown_ref, knownt_ref, idx0_ref, idx1_ref, idx2_ref, w_ref):
    b = pl.program_id(0)
    d = None
    for c in range(3):
        uc = unknown_ref[0, :, c:c + 1]
        kc = knownt_ref[0, c:c + 1, :]
        t = uc - kc
        d = t * t if d is None else d + t * t
    iota_f = lax.broadcasted_iota(jnp.int32, (TN, M), 1).astype(jnp.float32)
    dists, idxs = [], []
    for r in range(3):
        mj = jnp.min(d, axis=1, keepdims=True)
        msk = d <= mj
        ij = jnp.min(jnp.where(msk, iota_f, jnp.float32(M)), axis=1,
                     keepdims=True)
        if r < 2:
            d = jnp.where(msk, jnp.float32(jnp.inf), d)
        dists.append(mj)
        idxs.append(ij.astype(jnp.int32))
    recip = [1.0 / (dj + 1e-8) for dj in dists]
    norm = recip[0] + recip[1] + recip[2]
    w_ref[...] = jnp.concatenate(
        [r / norm for r in recip] + [jnp.zeros((TN, 5), jnp.float32)],
        axis=1)
    base = b * M
    idx0_ref[...] = idxs[0] + base
    idx1_ref[...] = idxs[1] + base
    idx2_ref[...] = idxs[2] + base


def _knn(unknown, knownt):
    bh = unknown.shape[0]
    pts = bh * N
    return pl.pallas_call(
        _knn_body,
        grid=(bh, NB),
        in_specs=[
            pl.BlockSpec((1, TN, 3), lambda b, n: (b, n, 0)),
            pl.BlockSpec((1, 3, M), lambda b, n: (b, 0, 0)),
        ],
        out_specs=[
            pl.BlockSpec((TN, 1), lambda b, n: (b * NB + n, 0)),
            pl.BlockSpec((TN, 1), lambda b, n: (b * NB + n, 0)),
            pl.BlockSpec((TN, 1), lambda b, n: (b * NB + n, 0)),
            pl.BlockSpec((TN, 8), lambda b, n: (b * NB + n, 0)),
        ],
        out_shape=[
            jax.ShapeDtypeStruct((pts, 1), jnp.int32),
            jax.ShapeDtypeStruct((pts, 1), jnp.int32),
            jax.ShapeDtypeStruct((pts, 1), jnp.int32),
            jax.ShapeDtypeStruct((pts, 8), jnp.float32),
        ],
    )(unknown, knownt)


_NC = 2
_NS = 16
_NW = _NC * _NS
_CH = 128


@functools.cache
def _sc_gather_kernel(pts):
    per_w = pts // _NW
    nch = per_w // _CH

    def body_fn(table, i0, i1, i2, g0, g1, g2,
                i0_v, i1_v, i2_v, r0_v, r1_v, r2_v, sem):
        wid = lax.axis_index("s") * _NC + lax.axis_index("c")
        base = wid * per_w

        def body(ci, carry):
            off = base + ci * _CH
            pltpu.sync_copy(i0.at[pl.ds(off, _CH)], i0_v)
            pltpu.sync_copy(i1.at[pl.ds(off, _CH)], i1_v)
            pltpu.sync_copy(i2.at[pl.ds(off, _CH)], i2_v)
            c0 = pltpu.async_copy(table.at[i0_v], r0_v, sem)
            c1 = pltpu.async_copy(table.at[i1_v], r1_v, sem)
            c2 = pltpu.async_copy(table.at[i2_v], r2_v, sem)
            c0.wait()
            c1.wait()
            c2.wait()
            pltpu.sync_copy(r0_v, g0.at[pl.ds(off, _CH)])
            pltpu.sync_copy(r1_v, g1.at[pl.ds(off, _CH)])
            pltpu.sync_copy(r2_v, g2.at[pl.ds(off, _CH)])
            return carry

        lax.fori_loop(0, nch, body, 0)

    return functools.partial(
        pl.kernel,
        mesh=plsc.VectorSubcoreMesh(core_axis_name="c", subcore_axis_name="s",
                                    num_cores=_NC, num_subcores=_NS),
        out_type=[jax.ShapeDtypeStruct((pts, C2), jnp.float32)] * 3,
        scratch_types=[
            pltpu.VMEM((_CH,), jnp.int32),
            pltpu.VMEM((_CH,), jnp.int32),
            pltpu.VMEM((_CH,), jnp.int32),
            pltpu.VMEM((_CH, C2), jnp.float32),
            pltpu.VMEM((_CH, C2), jnp.float32),
            pltpu.VMEM((_CH, C2), jnp.float32),
            pltpu.SemaphoreType.DMA,
        ],
    )(body_fn)


def _mlp_body(g0_ref, g1_ref, g2_ref, w_ref, uf_ref, W_ref, b_ref,
              y_ref, acc_ref):
    b = pl.program_id(0)
    nb = pl.program_id(1)
    w = w_ref[...]
    x1 = (g0_ref[...] * w[:, 0:1] + g1_ref[...] * w[:, 1:2]
          + g2_ref[...] * w[:, 2:3])
    uf = uf_ref[0]
    Wm = W_ref[...]
    y = lax.dot_general(Wm[:, :C2], x1, (((1,), (1,)), ((), ())),
                        preferred_element_type=jnp.float32)
    y = y + lax.dot_general(Wm[:, C2:], uf, (((1,), (0,)), ((), ())),
                            preferred_element_type=jnp.float32)
    y = y + b_ref[...]
    y_ref[0] = y
    part = jnp.concatenate([jnp.sum(y, axis=1, keepdims=True),
                            jnp.sum(y * y, axis=1, keepdims=True)], axis=1)

    @pl.when((b == 0) & (nb == 0))
    def _init():
        acc_ref[...] = jnp.zeros_like(acc_ref)

    acc_ref[...] += part


def _mlp(g0, g1, g2, wts, unknow_feats, W, b2):
    bh = unknow_feats.shape[0]
    return pl.pallas_call(
        _mlp_body,
        grid=(bh, NB),
        in_specs=[
            pl.BlockSpec((TN, C2), lambda b, n: (b * NB + n, 0)),
            pl.BlockSpec((TN, C2), lambda b, n: (b * NB + n, 0)),
            pl.BlockSpec((TN, C2), lambda b, n: (b * NB + n, 0)),
            pl.BlockSpec((TN, 8), lambda b, n: (b * NB + n, 0)),
            pl.BlockSpec((1, C1, TN), lambda b, n: (b, 0, n)),
            pl.BlockSpec((C_OUT, C_IN), lambda b, n: (0, 0)),
            pl.BlockSpec((C_OUT, 1), lambda b, n: (0, 0)),
        ],
        out_specs=[
            pl.BlockSpec((1, C_OUT, TN), lambda b, n: (b, 0, n)),
            pl.BlockSpec((C_OUT, 2), lambda b, n: (0, 0)),
        ],
        out_shape=[
            jax.ShapeDtypeStruct((bh, C_OUT, N), jnp.float32),
            jax.ShapeDtypeStruct((C_OUT, 2), jnp.float32),
        ],
    )(g0, g1, g2, wts, unknow_feats, W, b2)


def _bn_body(y_ref, acc_ref, gamma_ref, beta_ref, out_ref):
    cnt = jnp.float32(B * N)
    mean = acc_ref[:, 0:1] / cnt
    var = acc_ref[:, 1:2] / cnt - mean * mean
    scale = gamma_ref[...] * lax.rsqrt(var + 1e-5)
    shift = beta_ref[...] - mean * scale
    out_ref[0] = jnp.maximum(y_ref[0] * scale + shift, 0.0)


def _bn(y, acc, gamma2, beta2):
    bh = y.shape[0]
    return pl.pallas_call(
        _bn_body,
        grid=(bh, NB),
        in_specs=[
            pl.BlockSpec((1, C_OUT, TN), lambda b, n: (b, 0, n)),
            pl.BlockSpec((C_OUT, 2), lambda b, n: (0, 0)),
            pl.BlockSpec((C_OUT, 1), lambda b, n: (0, 0)),
            pl.BlockSpec((C_OUT, 1), lambda b, n: (0, 0)),
        ],
        out_specs=pl.BlockSpec((1, C_OUT, TN), lambda b, n: (b, 0, n)),
        out_shape=jax.ShapeDtypeStruct((bh, C_OUT, N), jnp.float32),
    )(y, acc, gamma2, beta2)


def kernel(unknown, known, unknow_feats, known_feats, W, b, gamma, beta):
    knownt = jnp.transpose(known, (0, 2, 1))
    table = jnp.transpose(known_feats, (0, 2, 1)).reshape(B * M, C2)
    b2 = b.reshape(C_OUT, 1)
    bh = B // 4
    pts = bh * N

    ys, accs = [], []
    for h in range(4):
        s = slice(h * bh, (h + 1) * bh)
        idx0, idx1, idx2, wts = _knn(unknown[s], knownt[s])
        g0, g1, g2 = _sc_gather_kernel(pts)(
            table[h * bh * M:(h + 1) * bh * M],
            idx0.reshape(pts), idx1.reshape(pts), idx2.reshape(pts))
        y, acc = _mlp(g0, g1, g2, wts, unknow_feats[s], W, b2)
        ys.append(y)
        accs.append(acc)

    acc = accs[0] + accs[1]
    gamma2 = gamma.reshape(C_OUT, 1)
    beta2 = beta.reshape(C_OUT, 1)
    return jnp.concatenate([_bn(y, acc, gamma2, beta2) for y in ys], axis=0)


# speedup vs baseline: 1.2132x; 1.2132x over previous
import functools

import jax
import jax.numpy as jnp
from jax import lax
from jax.experimental import pallas as pl
from jax.experimental.pallas import tpu as pltpu
from jax.experimental.pallas import tpu_sc as plsc

B, N, M = 8, 8192, 1024
C1, C2 = 64, 128
C_IN, C_OUT = 192, 128

TN = 4096
NB = N // TN


def _knn_body(unknown_ref, knownt_ref, idx0_ref, idx1_ref, idx2_ref, w_ref):
    b = pl.program_id(0)
    d = None
    for c in range(3):
        uc = unknown_ref[0, :, c:c + 1]
        kc = knownt_ref[0, c:c + 1, :]
        t = uc - kc
        d = t * t if d is None else d + t * t
    iota_f = lax.broadcasted_iota(jnp.int32, (TN, M), 1).astype(jnp.float32)
    dists, idxs = [], []
    for r in range(3):
        mj = jnp.min(d, axis=1, keepdims=True)
        msk = d <= mj
        ij = jnp.min(jnp.where(msk, iota_f, jnp.float32(M)), axis=1,
                     keepdims=True)
        if r < 2:
            d = jnp.where(msk, jnp.float32(jnp.inf), d)
        dists.append(mj)
        idxs.append(ij.astype(jnp.int32))
    recip = [1.0 / (dj + 1e-8) for dj in dists]
    norm = recip[0] + recip[1] + recip[2]
    wcols = jnp.concatenate(
        [r / norm for r in recip] + [jnp.zeros((TN, 5), jnp.float32)],
        axis=1)
    w_ref[0] = jnp.transpose(wcols, (1, 0))
    base = b * M
    icols = jnp.concatenate([i + base for i in idxs], axis=1)
    irows = jnp.transpose(icols, (1, 0))
    idx0_ref[0] = irows[0:1]
    idx1_ref[0] = irows[1:2]
    idx2_ref[0] = irows[2:3]


def _knn(unknown, knownt):
    bh = unknown.shape[0]
    nbt = bh * NB
    return pl.pallas_call(
        _knn_body,
        grid=(bh, NB),
        in_specs=[
            pl.BlockSpec((1, TN, 3), lambda b, n: (b, n, 0)),
            pl.BlockSpec((1, 3, M), lambda b, n: (b, 0, 0)),
        ],
        out_specs=[
            pl.BlockSpec((1, 1, TN), lambda b, n: (b * NB + n, 0, 0)),
            pl.BlockSpec((1, 1, TN), lambda b, n: (b * NB + n, 0, 0)),
            pl.BlockSpec((1, 1, TN), lambda b, n: (b * NB + n, 0, 0)),
            pl.BlockSpec((1, 8, TN), lambda b, n: (b * NB + n, 0, 0)),
        ],
        out_shape=[
            jax.ShapeDtypeStruct((nbt, 1, TN), jnp.int32),
            jax.ShapeDtypeStruct((nbt, 1, TN), jnp.int32),
            jax.ShapeDtypeStruct((nbt, 1, TN), jnp.int32),
            jax.ShapeDtypeStruct((nbt, 8, TN), jnp.float32),
        ],
    )(unknown, knownt)


_NC = 2
_NS = 16
_NW = _NC * _NS
_CH = 128


@functools.cache
def _sc_gather_kernel(pts):
    per_w = pts // _NW
    nch = per_w // _CH

    def body_fn(table, i0, i1, i2, g0, g1, g2,
                i0_v, i1_v, i2_v, r0_v, r1_v, r2_v, sem):
        wid = lax.axis_index("s") * _NC + lax.axis_index("c")
        base = wid * per_w

        def body(ci, carry):
            off = base + ci * _CH
            pltpu.sync_copy(i0.at[pl.ds(off, _CH)], i0_v)
            pltpu.sync_copy(i1.at[pl.ds(off, _CH)], i1_v)
            pltpu.sync_copy(i2.at[pl.ds(off, _CH)], i2_v)
            c0 = pltpu.async_copy(table.at[i0_v], r0_v, sem)
            c1 = pltpu.async_copy(table.at[i1_v], r1_v, sem)
            c2 = pltpu.async_copy(table.at[i2_v], r2_v, sem)
            c0.wait()
            c1.wait()
            c2.wait()
            pltpu.sync_copy(r0_v, g0.at[pl.ds(off, _CH)])
            pltpu.sync_copy(r1_v, g1.at[pl.ds(off, _CH)])
            pltpu.sync_copy(r2_v, g2.at[pl.ds(off, _CH)])
            return carry

        lax.fori_loop(0, nch, body, 0)

    return functools.partial(
        pl.kernel,
        mesh=plsc.VectorSubcoreMesh(core_axis_name="c", subcore_axis_name="s",
                                    num_cores=_NC, num_subcores=_NS),
        out_type=[jax.ShapeDtypeStruct((pts, C2), jnp.float32)] * 3,
        scratch_types=[
            pltpu.VMEM((_CH,), jnp.int32),
            pltpu.VMEM((_CH,), jnp.int32),
            pltpu.VMEM((_CH,), jnp.int32),
            pltpu.VMEM((_CH, C2), jnp.float32),
            pltpu.VMEM((_CH, C2), jnp.float32),
            pltpu.VMEM((_CH, C2), jnp.float32),
            pltpu.SemaphoreType.DMA,
        ],
    )(body_fn)


def _mlp_body(*refs):
    g0_ref, g1_ref, g2_ref, w_ref, uf_ref, W_ref, b_ref = refs[:7]
    y_ref, acc_ref = refs[-2:]
    b = pl.program_id(0)
    nb = pl.program_id(1)
    w = jnp.transpose(w_ref[0], (1, 0))
    x1 = (g0_ref[...] * w[:, 0:1] + g1_ref[...] * w[:, 1:2]
          + g2_ref[...] * w[:, 2:3])
    uf = uf_ref[0]
    Wm = W_ref[...]
    y = lax.dot_general(Wm[:, :C2], x1, (((1,), (1,)), ((), ())),
                        preferred_element_type=jnp.float32)
    y = y + lax.dot_general(Wm[:, C2:], uf, (((1,), (0,)), ((), ())),
                            preferred_element_type=jnp.float32)
    y = y + b_ref[...]
    y_ref[0] = y
    part = jnp.concatenate([jnp.sum(y, axis=1, keepdims=True),
                            jnp.sum(y * y, axis=1, keepdims=True)], axis=1)

    @pl.when((b == 0) & (nb == 0))
    def _init():
        acc_ref[...] = jnp.zeros_like(acc_ref)

    acc_ref[...] += part


def _mlp(g0, g1, g2, wts, unknow_feats, W, b2, boff, y_prev=None):
    bh = unknow_feats.shape[0]
    in_specs = [
        pl.BlockSpec((TN, C2), lambda b, n: (b * NB + n, 0)),
        pl.BlockSpec((TN, C2), lambda b, n: (b * NB + n, 0)),
        pl.BlockSpec((TN, C2), lambda b, n: (b * NB + n, 0)),
        pl.BlockSpec((1, 8, TN), lambda b, n: (b * NB + n, 0, 0)),
        pl.BlockSpec((1, C1, TN), lambda b, n: (b, 0, n)),
        pl.BlockSpec((C_OUT, C_IN), lambda b, n: (0, 0)),
        pl.BlockSpec((C_OUT, 1), lambda b, n: (0, 0)),
    ]
    args = [g0, g1, g2, wts, unknow_feats, W, b2]
    kwargs = {}
    if y_prev is not None:
        in_specs.append(pl.BlockSpec(memory_space=pl.ANY))
        args.append(y_prev)
        kwargs["input_output_aliases"] = {7: 0}
    return pl.pallas_call(
        _mlp_body,
        grid=(bh, NB),
        in_specs=in_specs,
        out_specs=[
            pl.BlockSpec((1, C_OUT, TN), lambda b, n, boff=boff: (b + boff, 0, n)),
            pl.BlockSpec((C_OUT, 2), lambda b, n: (0, 0)),
        ],
        out_shape=[
            jax.ShapeDtypeStruct((B, C_OUT, N), jnp.float32),
            jax.ShapeDtypeStruct((C_OUT, 2), jnp.float32),
        ],
        **kwargs,
    )(*args)


def _bn_body(y_ref, acc_ref, gamma_ref, beta_ref, out_ref):
    cnt = jnp.float32(B * N)
    mean = acc_ref[:, 0:1] / cnt
    var = acc_ref[:, 1:2] / cnt - mean * mean
    scale = gamma_ref[...] * lax.rsqrt(var + 1e-5)
    shift = beta_ref[...] - mean * scale
    out_ref[0] = jnp.maximum(y_ref[0] * scale + shift, 0.0)


def _bn(y, acc, gamma2, beta2):
    bh = y.shape[0]
    return pl.pallas_call(
        _bn_body,
        grid=(bh, NB),
        in_specs=[
            pl.BlockSpec((1, C_OUT, TN), lambda b, n: (b, 0, n)),
            pl.BlockSpec((C_OUT, 2), lambda b, n: (0, 0)),
            pl.BlockSpec((C_OUT, 1), lambda b, n: (0, 0)),
            pl.BlockSpec((C_OUT, 1), lambda b, n: (0, 0)),
        ],
        out_specs=pl.BlockSpec((1, C_OUT, TN), lambda b, n: (b, 0, n)),
        out_shape=jax.ShapeDtypeStruct((bh, C_OUT, N), jnp.float32),
    )(y, acc, gamma2, beta2)


def kernel(unknown, known, unknow_feats, known_feats, W, b, gamma, beta):
    knownt = jnp.transpose(known, (0, 2, 1))
    table = jnp.transpose(known_feats, (0, 2, 1)).reshape(B * M, C2)
    b2 = b.reshape(C_OUT, 1)
    bh = B // 2
    pts = bh * N

    y_full, accs = None, []
    for h in range(2):
        s = slice(h * bh, (h + 1) * bh)
        idx0, idx1, idx2, wts = _knn(unknown[s], knownt[s])
        g0, g1, g2 = _sc_gather_kernel(pts)(
            table[h * bh * M:(h + 1) * bh * M],
            idx0.reshape(pts), idx1.reshape(pts), idx2.reshape(pts))
        y_full, acc = _mlp(g0, g1, g2, wts, unknow_feats[s], W, b2,
                           h * bh, y_prev=y_full)
        accs.append(acc)

    acc = sum(accs)
    gamma2 = gamma.reshape(C_OUT, 1)
    beta2 = beta.reshape(C_OUT, 1)
    return _bn(y_full, acc, gamma2, beta2)


# speedup vs baseline: 1.2518x; 1.0318x over previous
import functools

import jax
import jax.numpy as jnp
from jax import lax
from jax.experimental import pallas as pl
from jax.experimental.pallas import tpu as pltpu
from jax.experimental.pallas import tpu_sc as plsc

B, N, M = 8, 8192, 1024
C1, C2 = 64, 128
C_IN, C_OUT = 192, 128

TN = 4096
NB = N // TN


def _knn_body(unknown_ref, knownt_ref, idx0_ref, idx1_ref, idx2_ref, w_ref,
              *, boff):
    b = pl.program_id(0) + boff
    d = None
    for c in range(3):
        uc = unknown_ref[0, :, c:c + 1]
        kc = knownt_ref[0, c:c + 1, :]
        t = uc - kc
        d = t * t if d is None else d + t * t
    iota_f = lax.broadcasted_iota(jnp.int32, (TN, M), 1).astype(jnp.float32)
    dists, idxs = [], []
    for r in range(3):
        mj = jnp.min(d, axis=1, keepdims=True)
        msk = d <= mj
        ij = jnp.min(jnp.where(msk, iota_f, jnp.float32(M)), axis=1,
                     keepdims=True)
        if r < 2:
            d = jnp.where(msk, jnp.float32(jnp.inf), d)
        dists.append(mj)
        idxs.append(ij.astype(jnp.int32))
    recip = [1.0 / (dj + 1e-8) for dj in dists]
    norm = recip[0] + recip[1] + recip[2]
    wcols = jnp.concatenate(
        [r / norm for r in recip] + [jnp.zeros((TN, 5), jnp.float32)],
        axis=1)
    w_ref[0] = jnp.transpose(wcols, (1, 0))
    base = b * M
    icols = jnp.concatenate([i + base for i in idxs], axis=1)
    irows = jnp.transpose(icols, (1, 0))
    idx0_ref[0] = irows[0:1]
    idx1_ref[0] = irows[1:2]
    idx2_ref[0] = irows[2:3]


def _knn(unknown, knownt, bh, boff):
    nbt = bh * NB
    return pl.pallas_call(
        functools.partial(_knn_body, boff=boff),
        grid=(bh, NB),
        in_specs=[
            pl.BlockSpec((1, TN, 3), lambda b, n: (b + boff, n, 0)),
            pl.BlockSpec((1, 3, M), lambda b, n: (b + boff, 0, 0)),
        ],
        out_specs=[
            pl.BlockSpec((1, 1, TN), lambda b, n: (b * NB + n, 0, 0)),
            pl.BlockSpec((1, 1, TN), lambda b, n: (b * NB + n, 0, 0)),
            pl.BlockSpec((1, 1, TN), lambda b, n: (b * NB + n, 0, 0)),
            pl.BlockSpec((1, 8, TN), lambda b, n: (b * NB + n, 0, 0)),
        ],
        out_shape=[
            jax.ShapeDtypeStruct((nbt, 1, TN), jnp.int32),
            jax.ShapeDtypeStruct((nbt, 1, TN), jnp.int32),
            jax.ShapeDtypeStruct((nbt, 1, TN), jnp.int32),
            jax.ShapeDtypeStruct((nbt, 8, TN), jnp.float32),
        ],
    )(unknown, knownt)


_NC = 2
_NS = 16
_NW = _NC * _NS
_CH = 128


@functools.cache
def _sc_gather_kernel(pts):
    per_w = pts // _NW
    nch = per_w // _CH

    def body_fn(table, i0, i1, i2, g0, g1, g2,
                i0_v, i1_v, i2_v, r0_v, r1_v, r2_v, sem):
        wid = lax.axis_index("s") * _NC + lax.axis_index("c")
        base = wid * per_w

        def body(ci, carry):
            off = base + ci * _CH
            pltpu.sync_copy(i0.at[pl.ds(off, _CH)], i0_v)
            pltpu.sync_copy(i1.at[pl.ds(off, _CH)], i1_v)
            pltpu.sync_copy(i2.at[pl.ds(off, _CH)], i2_v)
            c0 = pltpu.async_copy(table.at[i0_v], r0_v, sem)
            c1 = pltpu.async_copy(table.at[i1_v], r1_v, sem)
            c2 = pltpu.async_copy(table.at[i2_v], r2_v, sem)
            c0.wait()
            c1.wait()
            c2.wait()
            pltpu.sync_copy(r0_v, g0.at[pl.ds(off, _CH)])
            pltpu.sync_copy(r1_v, g1.at[pl.ds(off, _CH)])
            pltpu.sync_copy(r2_v, g2.at[pl.ds(off, _CH)])
            return carry

        lax.fori_loop(0, nch, body, 0)

    return functools.partial(
        pl.kernel,
        mesh=plsc.VectorSubcoreMesh(core_axis_name="c", subcore_axis_name="s",
                                    num_cores=_NC, num_subcores=_NS),
        out_type=[jax.ShapeDtypeStruct((pts, C2), jnp.float32)] * 3,
        scratch_types=[
            pltpu.VMEM((_CH,), jnp.int32),
            pltpu.VMEM((_CH,), jnp.int32),
            pltpu.VMEM((_CH,), jnp.int32),
            pltpu.VMEM((_CH, C2), jnp.float32),
            pltpu.VMEM((_CH, C2), jnp.float32),
            pltpu.VMEM((_CH, C2), jnp.float32),
            pltpu.SemaphoreType.DMA,
        ],
    )(body_fn)


def _mlp_body(*refs):
    g0_ref, g1_ref, g2_ref, w_ref, uf_ref, W_ref, b_ref = refs[:7]
    y_ref, acc_ref = refs[-2:]
    b = pl.program_id(0)
    nb = pl.program_id(1)
    w = jnp.transpose(w_ref[0], (1, 0))
    x1 = (g0_ref[...] * w[:, 0:1] + g1_ref[...] * w[:, 1:2]
          + g2_ref[...] * w[:, 2:3])
    uf = uf_ref[0]
    Wm = W_ref[...]
    y = lax.dot_general(Wm[:, :C2], x1, (((1,), (1,)), ((), ())),
                        preferred_element_type=jnp.float32)
    y = y + lax.dot_general(Wm[:, C2:], uf, (((1,), (0,)), ((), ())),
                            preferred_element_type=jnp.float32)
    y = y + b_ref[...]
    y_ref[0] = y
    part = jnp.concatenate([jnp.sum(y, axis=1, keepdims=True),
                            jnp.sum(y * y, axis=1, keepdims=True)], axis=1)

    @pl.when((b == 0) & (nb == 0))
    def _init():
        acc_ref[...] = jnp.zeros_like(acc_ref)

    acc_ref[...] += part


def _mlp(g0, g1, g2, wts, unknow_feats, W, b2, bh, boff, y_prev=None):
    in_specs = [
        pl.BlockSpec((TN, C2), lambda b, n: (b * NB + n, 0)),
        pl.BlockSpec((TN, C2), lambda b, n: (b * NB + n, 0)),
        pl.BlockSpec((TN, C2), lambda b, n: (b * NB + n, 0)),
        pl.BlockSpec((1, 8, TN), lambda b, n: (b * NB + n, 0, 0)),
        pl.BlockSpec((1, C1, TN), lambda b, n: (b + boff, 0, n)),
        pl.BlockSpec((C_OUT, C_IN), lambda b, n: (0, 0)),
        pl.BlockSpec((C_OUT, 1), lambda b, n: (0, 0)),
    ]
    args = [g0, g1, g2, wts, unknow_feats, W, b2]
    kwargs = {}
    if y_prev is not None:
        in_specs.append(pl.BlockSpec(memory_space=pl.ANY))
        args.append(y_prev)
        kwargs["input_output_aliases"] = {7: 0}
    return pl.pallas_call(
        _mlp_body,
        grid=(bh, NB),
        in_specs=in_specs,
        out_specs=[
            pl.BlockSpec((1, C_OUT, TN), lambda b, n, boff=boff: (b + boff, 0, n)),
            pl.BlockSpec((C_OUT, 2), lambda b, n: (0, 0)),
        ],
        out_shape=[
            jax.ShapeDtypeStruct((B, C_OUT, N), jnp.float32),
            jax.ShapeDtypeStruct((C_OUT, 2), jnp.float32),
        ],
        **kwargs,
    )(*args)


def _bn_body(y_ref, acc_ref, gamma_ref, beta_ref, out_ref):
    cnt = jnp.float32(B * N)
    mean = acc_ref[:, 0:1] / cnt
    var = acc_ref[:, 1:2] / cnt - mean * mean
    scale = gamma_ref[...] * lax.rsqrt(var + 1e-5)
    shift = beta_ref[...] - mean * scale
    out_ref[0] = jnp.maximum(y_ref[0] * scale + shift, 0.0)


def _bn(y, acc, gamma2, beta2):
    bh = y.shape[0]
    return pl.pallas_call(
        _bn_body,
        grid=(bh, NB),
        in_specs=[
            pl.BlockSpec((1, C_OUT, TN), lambda b, n: (b, 0, n)),
            pl.BlockSpec((C_OUT, 2), lambda b, n: (0, 0)),
            pl.BlockSpec((C_OUT, 1), lambda b, n: (0, 0)),
            pl.BlockSpec((C_OUT, 1), lambda b, n: (0, 0)),
        ],
        out_specs=pl.BlockSpec((1, C_OUT, TN), lambda b, n: (b, 0, n)),
        out_shape=jax.ShapeDtypeStruct((bh, C_OUT, N), jnp.float32),
    )(y, acc, gamma2, beta2)


def kernel(unknown, known, unknow_feats, known_feats, W, b, gamma, beta):
    knownt = jnp.transpose(known, (0, 2, 1))
    table = jnp.transpose(known_feats, (0, 2, 1)).reshape(B * M, C2)
    b2 = b.reshape(C_OUT, 1)
    bh = B // 2
    pts = bh * N

    y_full, accs = None, []
    for h in range(2):
        idx0, idx1, idx2, wts = _knn(unknown, knownt, bh, h * bh)
        g0, g1, g2 = _sc_gather_kernel(pts)(
            table, idx0.reshape(pts), idx1.reshape(pts), idx2.reshape(pts))
        y_full, acc = _mlp(g0, g1, g2, wts, unknow_feats, W, b2,
                           bh, h * bh, y_prev=y_full)
        accs.append(acc)

    acc = sum(accs)
    gamma2 = gamma.reshape(C_OUT, 1)
    beta2 = beta.reshape(C_OUT, 1)
    return _bn(y_full, acc, gamma2, beta2)
